# Initial kernel scaffold; baseline (speedup 1.0000x reference)
import jax
import jax.numpy as jnp
from jax.experimental import pallas as pl


def kernel(inputs, params):
    raise NotImplementedError("write your pallas kernel here")


# speedup vs baseline: 23.5052x; 23.5052x over previous
import functools

import jax
import jax.numpy as jnp
from jax.experimental import pallas as pl
from jax.experimental.pallas import tpu as pltpu

_F32 = jnp.float32

_EF4 = ((1.0, 0.0), (-1.0, 0.0), (0.0, 1.0), (0.0, -1.0))


def _gru_core(nx, h, hid, w4_ref, nnb_ref, gwih_ref, gbih_ref, gwhh_ref,
              gbhh_ref):
    n = hid.shape[0]
    row = jax.lax.broadcasted_iota(jnp.int32, (n, 1), 0)
    j = row % nx
    i = (row // nx) % nx
    m0 = (j >= 1)
    m1 = (j <= nx - 2)
    m2 = (i >= 1)
    m3 = (i <= nx - 2)
    z1 = jnp.zeros((1, h), _F32)
    znx = jnp.zeros((nx, h), _F32)
    s0 = jnp.where(m0, jnp.concatenate([z1, hid[:-1]], axis=0), 0.0)
    s1 = jnp.where(m1, jnp.concatenate([hid[1:], z1], axis=0), 0.0)
    s2 = jnp.where(m2, jnp.concatenate([znx, hid[:-nx]], axis=0), 0.0)
    s3 = jnp.where(m3, jnp.concatenate([hid[nx:], znx], axis=0), 0.0)
    xcat = jnp.concatenate([s0, s1, s2, s3], axis=1)
    ssum = jnp.dot(xcat, w4_ref[...], preferred_element_type=_F32)
    deg = (m0.astype(_F32) + m1.astype(_F32) + m2.astype(_F32)
           + m3.astype(_F32))
    m = jnp.maximum(ssum / deg + nnb_ref[...], 0.0)
    gi = jnp.dot(m, gwih_ref[...], preferred_element_type=_F32) + gbih_ref[...]
    gh = jnp.dot(hid, gwhh_ref[...], preferred_element_type=_F32) + gbhh_ref[...]
    r = jax.nn.sigmoid(gi[:, :h] + gh[:, :h])
    z = jax.nn.sigmoid(gi[:, h:2 * h] + gh[:, h:2 * h])
    nn = jnp.tanh(gi[:, 2 * h:] + r * gh[:, 2 * h:])
    return (1.0 - z) * nn + z * hid


def _mpnn_kern(nx, h, x_ref, pw1_ref, pb1_ref, pw2_ref, pb2_ref, w4_ref,
               nnb_ref, gwih_ref, gbih_ref, gwhh_ref, gbhh_ref, o_ref):
    l1 = jnp.maximum(
        jnp.dot(x_ref[...], pw1_ref[...], preferred_element_type=_F32)
        + pb1_ref[...], 0.0)
    hid = jnp.dot(l1, pw2_ref[...], preferred_element_type=_F32) + pb2_ref[...]
    o_ref[...] = _gru_core(nx, h, hid, w4_ref, nnb_ref, gwih_ref, gbih_ref,
                           gwhh_ref, gbhh_ref)


def _mpnn2_kern(nx, h, bp_ref, ur_ref, wa_ref, wb_ref, b1_ref, pw2_ref,
                pb2_ref, w4_ref, nnb_ref, gwih_ref, gbih_ref, gwhh_ref,
                gbhh_ref, o_ref):
    pre = (jnp.dot(bp_ref[...], wa_ref[...], preferred_element_type=_F32)
           + jnp.dot(ur_ref[...], wb_ref[...], preferred_element_type=_F32)
           + b1_ref[...])
    l1 = jnp.maximum(pre, 0.0)
    hid = jnp.dot(l1, pw2_ref[...], preferred_element_type=_F32) + pb2_ref[...]
    o_ref[...] = _gru_core(nx, h, hid, w4_ref, nnb_ref, gwih_ref, gbih_ref,
                           gwhh_ref, gbhh_ref)


def _edge_w4(p, h):
    ef = jnp.asarray(_EF4, _F32)
    a = jnp.maximum(ef @ p['eW1'] + p['eb1'], 0.0)
    w = (a @ p['eW2'] + p['eb2']).reshape(4, h, h)
    return w.reshape(4 * h, h)


def _row(v):
    return v.reshape(1, -1)


def _mpnn_call(x, nx, h, p):
    n = x.shape[0]
    fn = functools.partial(_mpnn_kern, nx, h)
    return pl.pallas_call(
        fn,
        out_shape=jax.ShapeDtypeStruct((n, h), _F32),
    )(x, p['pW1'], _row(p['pb1']), p['pW2'], _row(p['pb2']), _edge_w4(p, h),
      _row(p['nnb']), p['gWih'], _row(p['gbih']), p['gWhh'], _row(p['gbhh']))


def _mpnn2_call(bp, ur, nx, h, p, up_w, up_b):
    n = bp.shape[0]
    wa = p['pW1'][:h]
    wb = up_w @ p['pW1'][h:]
    b1 = p['pb1'] + up_b @ p['pW1'][h:]
    fn = functools.partial(_mpnn2_kern, nx, h)
    return pl.pallas_call(
        fn,
        out_shape=jax.ShapeDtypeStruct((n, h), _F32),
    )(bp, ur, wa, wb, _row(b1), p['pW2'], _row(p['pb2']), _edge_w4(p, h),
      _row(p['nnb']), p['gWih'], _row(p['gbih']), p['gWhh'], _row(p['gbhh']))


def kernel(inputs, params):
    b, t, nx, ny, c = inputs.shape
    h1 = params['c1']['pb2'].shape[0]
    h2 = params['lw']['pb2'].shape[0]
    outs = []
    for bi in range(b):
        x = inputs[bi].reshape(t * nx * ny, c)
        bp = _mpnn_call(x, nx, h1, params['c1'])
        d = bp.reshape(t, nx // 2, 2, ny // 2, 2, h1).mean(axis=(2, 4))
        hh = _mpnn_call(d.reshape(t * (nx // 2) * (ny // 2), h1), nx // 2,
                        h2, params['lw'])
        u = hh.reshape(t, nx // 2, ny // 2, h2)
        u = jnp.repeat(jnp.repeat(u, 2, axis=1), 2, axis=2)
        h3 = _mpnn2_call(bp, u.reshape(t * nx * ny, h2), nx, h1,
                         params['c2'], params['upW'], params['upb'])
        outs.append(h3.reshape(t, nx, ny, h1))
    return jnp.stack(outs, 0)


# speedup vs baseline: 36.1195x; 1.5367x over previous
import functools

import jax
import jax.numpy as jnp
from jax.experimental import pallas as pl
from jax.experimental.pallas import tpu as pltpu

_F32 = jnp.float32

_EF4 = ((1.0, 0.0), (-1.0, 0.0), (0.0, 1.0), (0.0, -1.0))


def _gru_core(nx, h, hid, w4, nnb, gwih, gbih, gwhh, gbhh):
    n = hid.shape[0]
    row = jax.lax.broadcasted_iota(jnp.int32, (n, 1), 0)
    j = row % nx
    i = (row // nx) % nx
    m0 = (j >= 1)
    m1 = (j <= nx - 2)
    m2 = (i >= 1)
    m3 = (i <= nx - 2)
    z1 = jnp.zeros((1, h), _F32)
    znx = jnp.zeros((nx, h), _F32)
    s0 = jnp.where(m0, jnp.concatenate([z1, hid[:-1]], axis=0), 0.0)
    s1 = jnp.where(m1, jnp.concatenate([hid[1:], z1], axis=0), 0.0)
    s2 = jnp.where(m2, jnp.concatenate([znx, hid[:-nx]], axis=0), 0.0)
    s3 = jnp.where(m3, jnp.concatenate([hid[nx:], znx], axis=0), 0.0)
    xcat = jnp.concatenate([s0, s1, s2, s3], axis=1)
    ssum = jnp.dot(xcat, w4, preferred_element_type=_F32)
    deg = (m0.astype(_F32) + m1.astype(_F32) + m2.astype(_F32)
           + m3.astype(_F32))
    m = jnp.maximum(ssum / deg + nnb, 0.0)
    gi = jnp.dot(m, gwih, preferred_element_type=_F32) + gbih
    gh = jnp.dot(hid, gwhh, preferred_element_type=_F32) + gbhh
    r = jax.nn.sigmoid(gi[:, :h] + gh[:, :h])
    z = jax.nn.sigmoid(gi[:, h:2 * h] + gh[:, h:2 * h])
    nn = jnp.tanh(gi[:, 2 * h:] + r * gh[:, 2 * h:])
    return (1.0 - z) * nn + z * hid


def _unet_kern(t, nx, h1, h2,
               x_ref,
               aw1, ab1, aw2, ab2, a4, anb, awih, abih, awhh, abhh,
               bw1, bb1, bw2, bb2, b4, bnb, bwih, bbih, bwhh, bbhh,
               cwa, cwb, cb1, cw2, cb2, c4, cnb, cwih, cbih, cwhh, cbhh,
               o_ref, spool_ref, sup_ref):
    nf = t * nx * nx
    nh = nx // 2
    nc = t * nh * nh

    l1 = jnp.maximum(
        jnp.dot(x_ref[...], aw1[...], preferred_element_type=_F32)
        + ab1[...], 0.0)
    hid = jnp.dot(l1, aw2[...], preferred_element_type=_F32) + ab2[...]
    bp = _gru_core(nx, h1, hid, a4[...], anb[...], awih[...], abih[...],
                   awhh[...], abhh[...])

    z1 = jnp.zeros((1, h1), _F32)
    spool_ref[...] = bp + jnp.concatenate([bp[1:], z1], axis=0)
    t1 = spool_ref[pl.Slice(0, nf // 2, 2), :]
    t4 = t1.reshape(t * nx // 2, 2, nh, h1)
    d = ((t4[:, 0] + t4[:, 1]) * 0.25).reshape(nc, h1)

    l1b = jnp.maximum(
        jnp.dot(d, bw1[...], preferred_element_type=_F32) + bb1[...], 0.0)
    hidb = jnp.dot(l1b, bw2[...], preferred_element_type=_F32) + bb2[...]
    h2v = _gru_core(nh, h2, hidb, b4[...], bnb[...], bwih[...], bbih[...],
                    bwhh[...], bbhh[...])

    u3 = h2v.reshape(t * nh, 1, nh, h2)
    ui = jnp.concatenate([u3, u3], axis=1).reshape(nf // 2, h2)
    v = jnp.dot(ui, cwb[...], preferred_element_type=_F32)
    sup_ref[pl.Slice(0, nf // 2, 2), :] = v
    sup_ref[pl.Slice(1, nf // 2, 2), :] = v

    pre = (jnp.dot(bp, cwa[...], preferred_element_type=_F32)
           + sup_ref[...]
           + cb1[...])
    l1c = jnp.maximum(pre, 0.0)
    hidc = jnp.dot(l1c, cw2[...], preferred_element_type=_F32) + cb2[...]
    o_ref[...] = _gru_core(nx, h1, hidc, c4[...], cnb[...], cwih[...],
                           cbih[...], cwhh[...], cbhh[...])


def _edge_w4(p, h):
    ef = jnp.asarray(_EF4, _F32)
    a = jnp.maximum(ef @ p['eW1'] + p['eb1'], 0.0)
    w = (a @ p['eW2'] + p['eb2']).reshape(4, h, h)
    return w.reshape(4 * h, h)


def _row(v):
    return v.reshape(1, -1)


def _block_args(p, h):
    return (p['pW1'], _row(p['pb1']), p['pW2'], _row(p['pb2']),
            _edge_w4(p, h), _row(p['nnb']), p['gWih'], _row(p['gbih']),
            p['gWhh'], _row(p['gbhh']))


def kernel(inputs, params):
    b, t, nx, ny, c = inputs.shape
    h1 = params['c1']['pb2'].shape[0]
    h2 = params['lw']['pb2'].shape[0]
    p2 = params['c2']
    cwa = p2['pW1'][:h1]
    cwb = params['upW'] @ p2['pW1'][h1:]
    cb1 = p2['pb1'] + params['upb'] @ p2['pW1'][h1:]
    c2_args = (cwa, cwb, _row(cb1), p2['pW2'], _row(p2['pb2']),
               _edge_w4(p2, h1), _row(p2['nnb']), p2['gWih'],
               _row(p2['gbih']), p2['gWhh'], _row(p2['gbhh']))
    fn = functools.partial(_unet_kern, t, nx, h1, h2)
    nf = t * nx * ny
    call = pl.pallas_call(
        fn,
        out_shape=jax.ShapeDtypeStruct((nf, h1), _F32),
        scratch_shapes=[pltpu.VMEM((nf, h1), _F32),
                        pltpu.VMEM((nf, h1), _F32)],
    )
    outs = []
    for bi in range(b):
        x = inputs[bi].reshape(t * nx * ny, c)
        h3 = call(x, *_block_args(params['c1'], h1),
                  *_block_args(params['lw'], h2), *c2_args)
        outs.append(h3.reshape(t, nx, ny, h1))
    return jnp.stack(outs, 0)
